# Initial kernel scaffold; baseline (speedup 1.0000x reference)
import jax
import jax.numpy as jnp
from jax.experimental import pallas as pl


def kernel(x, source, destination, normalization, relation_weights, self_loop_W, ln1_g, ln1_b, ln2_g, ln2_b, ffn_in_W, ffn_out_W):
    raise NotImplementedError("write your pallas kernel here")


# speedup vs baseline: 1.8262x; 1.8262x over previous
import functools

import jax
import jax.numpy as jnp
from jax import lax
from jax.experimental import pallas as pl
from jax.experimental.pallas import tpu as pltpu
from jax.experimental.pallas import tpu_sc as plsc

_N = 10000
_D = 128
_R = 8
_E = 40000
_INNER = 512
_EPS = 1e-5

_NS = 16
_EPW = (_R * _E) // _NS
_EB = 80
_NCHUNK = _EPW // _EB
_HALF = 5120
_AGG = _HALF + 8
_ROWS_PW = _HALF // _NS
_BN = 400


def _gelu(h):
    return 0.5 * h * (1.0 + lax.erf(h * 0.7071067811865476))


def _ln(h, g, b):
    m = jnp.mean(h, axis=-1, keepdims=True)
    v = jnp.mean((h - m) ** 2, axis=-1, keepdims=True)
    return (h - m) * lax.rsqrt(v + _EPS) * g + b


def _transform_body(x_ref, w_ref, o_ref):
    r = pl.program_id(0)
    xb = x_ref[...]
    acc = jnp.dot(xb, w_ref[0], preferred_element_type=jnp.float32)

    @pl.when(r == _R)
    def _():
        o_ref[0] = acc + xb

    @pl.when(r != _R)
    def _():
        o_ref[0] = acc


def _transform(x, w9):
    return pl.pallas_call(
        _transform_body,
        grid=(_R + 1, _N // _BN),
        in_specs=[
            pl.BlockSpec((_BN, _D), lambda r, i: (i, 0)),
            pl.BlockSpec((1, _D, _D), lambda r, i: (r, 0, 0)),
        ],
        out_specs=pl.BlockSpec((1, _BN, _D), lambda r, i: (r, i, 0)),
        out_shape=jax.ShapeDtypeStruct((_R + 1, _N, _D), jnp.float32),
    )(x, w9)


def _sc_body(table_hbm, src_hbm, dst_hbm, nrm_hbm, out_hbm,
             agg, zbuf, rows, idx_v, dst_v, nrm_v, sem):
    cid = lax.axis_index("c")
    sid = lax.axis_index("s")
    roff = (sid // 2) * _N
    doff = cid * _HALF

    zero = jnp.zeros((16,), jnp.float32)

    def _zfill(j, carry):
        for k in range(8):
            zbuf[j, pl.ds(k * 16, 16)] = zero
        return carry

    lax.fori_loop(0, _ROWS_PW + 8, _zfill, 0)
    pltpu.sync_copy(zbuf.at[pl.ds(0, _ROWS_PW)],
                    agg.at[pl.ds(sid * _ROWS_PW, _ROWS_PW)])

    @pl.when(sid == 0)
    def _zdump():
        pltpu.sync_copy(zbuf.at[pl.ds(_ROWS_PW, 8)], agg.at[pl.ds(_HALF, 8)])

    plsc.subcore_barrier()

    base_w = sid * _EPW
    dnums = lax.GatherDimensionNumbers(
        offset_dims=(), collapsed_slice_dims=(0,), start_index_map=(0,))

    def _chunk(c, carry):
        base = base_w + c * _EB
        pltpu.sync_copy(src_hbm.at[pl.ds(base, _EB)], idx_v)
        pltpu.sync_copy(dst_hbm.at[pl.ds(base, _EB)], dst_v)
        pltpu.sync_copy(nrm_hbm.at[pl.ds(base, _EB)], nrm_v)
        for k in range(_EB // 16):
            sl = pl.ds(k * 16, 16)
            idx_v[sl] = idx_v[sl] + roff
            t = dst_v[sl] - doff
            ok = (t >= 0) & (t < _HALF)
            dst_v[sl] = jnp.where(ok, t, _HALF)
        pltpu.async_copy(table_hbm.at[idx_v], rows, sem).wait()

        def _scale(g, inner):
            nv16 = nrm_v[pl.ds(g * 16, 16)]
            for i in range(16):
                nv = lax.gather(
                    nv16, jnp.full((16, 1), i, jnp.int32), dnums,
                    slice_sizes=(1,),
                    mode=lax.GatherScatterMode.PROMISE_IN_BOUNDS)
                j = g * 16 + i
                for k in range(8):
                    sl = pl.ds(k * 16, 16)
                    rows[j, sl] = rows[j, sl] * nv
            return inner

        lax.fori_loop(0, _EB // 16, _scale, 0)
        pltpu.sync_copy(rows, agg.at[dst_v], add=True)
        return carry

    lax.fori_loop(0, _NCHUNK, _chunk, 0)
    plsc.subcore_barrier()

    sl = pl.ds(sid * _ROWS_PW, _ROWS_PW)
    pltpu.sync_copy(agg.at[sl], out_hbm.at[cid, sl])


def _aggregate(table, src, dst, nrm):
    mesh = plsc.VectorSubcoreMesh(core_axis_name="c", subcore_axis_name="s")
    fn = pl.kernel(
        _sc_body,
        out_type=jax.ShapeDtypeStruct((2, _HALF, _D), jnp.float32),
        mesh=mesh,
        scratch_types=[
            pltpu.VMEM_SHARED((_AGG, _D), jnp.float32),
            pltpu.VMEM((_ROWS_PW + 8, _D), jnp.float32),
            pltpu.VMEM((_EB, _D), jnp.float32),
            pltpu.VMEM((_EB,), jnp.int32),
            pltpu.VMEM((_EB,), jnp.int32),
            pltpu.VMEM((_EB,), jnp.float32),
            pltpu.SemaphoreType.DMA,
        ],
    )
    return fn(table, src, dst, nrm)


def _epilogue_body(p_ref, s_ref, g1_ref, b1_ref, g2_ref, b2_ref,
                   fin_ref, fout_ref, o_ref):
    t = p_ref[...] + s_ref[...]
    h = _gelu(_ln(t, g1_ref[...], b1_ref[...]))
    y = jnp.dot(_ln(h, g2_ref[...], b2_ref[...]), fin_ref[...],
                preferred_element_type=jnp.float32)
    y = jnp.dot(_gelu(y), fout_ref[...], preferred_element_type=jnp.float32)
    o_ref[...] = h + y


def _epilogue(partials, selfpart, g1, b1, g2, b2, fin_t, fout_t):
    vec = pl.BlockSpec((1, _D), lambda i: (0, 0))
    return pl.pallas_call(
        _epilogue_body,
        grid=(_N // _BN,),
        in_specs=[
            pl.BlockSpec((_BN, _D), lambda i: (i, 0)),
            pl.BlockSpec((_BN, _D), lambda i: (i, 0)),
            vec, vec, vec, vec,
            pl.BlockSpec((_D, _INNER), lambda i: (0, 0)),
            pl.BlockSpec((_INNER, _D), lambda i: (0, 0)),
        ],
        out_specs=pl.BlockSpec((_BN, _D), lambda i: (i, 0)),
        out_shape=jax.ShapeDtypeStruct((_N, _D), jnp.float32),
    )(partials, selfpart, g1, b1, g2, b2, fin_t, fout_t)


def kernel(x, source, destination, normalization, relation_weights,
           self_loop_W, ln1_g, ln1_b, ln2_g, ln2_b, ffn_in_W, ffn_out_W):
    w9 = jnp.concatenate([relation_weights, self_loop_W.T[None]], axis=0)
    t9 = _transform(x, w9)
    selfpart = t9[_R]
    partials = _aggregate(
        t9[:_R].reshape(_R * _N, _D),
        source.reshape(-1),
        destination.reshape(-1),
        normalization.reshape(-1),
    )
    return _epilogue(
        partials.reshape(2 * _HALF, _D), selfpart,
        ln1_g.reshape(1, _D), ln1_b.reshape(1, _D),
        ln2_g.reshape(1, _D), ln2_b.reshape(1, _D),
        ffn_in_W.T, ffn_out_W.T,
    )


# speedup vs baseline: 2.8353x; 1.5526x over previous
import jax
import jax.numpy as jnp
from jax import lax
from jax.experimental import pallas as pl
from jax.experimental.pallas import tpu as pltpu
from jax.experimental.pallas import tpu_sc as plsc

_N = 10000
_D = 128
_R = 8
_E = 40000
_INNER = 512
_EPS = 1e-5

_NS = 16
_EPW = (_R * _E) // _NS
_EB = 80
_GB = 2
_GE = _EB * _GB
_NGRP = _EPW // _GE
_HALF = 5120
_AGG = _HALF + 8
_ROWS_PW = _HALF // _NS
_BN = 400


def _gelu(h):
    return 0.5 * h * (1.0 + lax.erf(h * 0.7071067811865476))


def _ln(h, g, b):
    m = jnp.mean(h, axis=-1, keepdims=True)
    v = jnp.mean((h - m) ** 2, axis=-1, keepdims=True)
    return (h - m) * lax.rsqrt(v + _EPS) * g + b


def _transform_body(x_ref, w_ref, tab_ref, self_ref):
    r = pl.program_id(1)
    xb = x_ref[...]
    acc = jnp.dot(xb, w_ref[0], preferred_element_type=jnp.float32)

    @pl.when(r < _R)
    def _():
        tab_ref[0] = acc

    @pl.when(r == _R)
    def _():
        self_ref[...] = acc + xb


def _transform(x, w9):
    return pl.pallas_call(
        _transform_body,
        grid=(_N // _BN, _R + 1),
        in_specs=[
            pl.BlockSpec((_BN, _D), lambda i, r: (i, 0)),
            pl.BlockSpec((1, _D, _D), lambda i, r: (r, 0, 0)),
        ],
        out_specs=[
            pl.BlockSpec((1, _BN, _D), lambda i, r: (jnp.minimum(r, _R - 1), i, 0)),
            pl.BlockSpec((_BN, _D), lambda i, r: (i, 0)),
        ],
        out_shape=[
            jax.ShapeDtypeStruct((_R, _N, _D), jnp.float32),
            jax.ShapeDtypeStruct((_N, _D), jnp.float32),
        ],
    )(x, w9)


_DNUMS = lax.GatherDimensionNumbers(
    offset_dims=(), collapsed_slice_dims=(0,), start_index_map=(0,))


def _sc_body(table_hbm, src_hbm, dst_hbm, nrm_hbm, out_hbm, *scr):
    agg, zbuf, rows = scr[0], scr[1], scr[2]
    ng = 2 * _GB
    idx = (scr[3:3 + _GB], scr[3 + _GB:3 + ng])
    dstv = (scr[3 + ng:3 + ng + _GB], scr[3 + ng + _GB:3 + 2 * ng])
    nrm_m = (scr[3 + 2 * ng], scr[4 + 2 * ng])
    gsem = (scr[5 + 2 * ng], scr[6 + 2 * ng])
    ssem = (scr[7 + 2 * ng], scr[8 + 2 * ng])
    cid = lax.axis_index("c")
    sid = lax.axis_index("s")
    roff = (sid // 2) * _N
    doff = cid * _HALF
    base_w = sid * _EPW

    zero = jnp.zeros((16,), jnp.float32)

    def _zfill(j, carry):
        for k in range(8):
            zbuf[j, pl.ds(k * 16, 16)] = zero
        return carry

    lax.fori_loop(0, 40, _zfill, 0)
    for z in range(_ROWS_PW // 40):
        pltpu.sync_copy(zbuf, agg.at[pl.ds(sid * _ROWS_PW + z * 40, 40)])

    @pl.when(sid == 0)
    def _zdump():
        pltpu.sync_copy(zbuf.at[pl.ds(0, 8)], agg.at[pl.ds(_HALF, 8)])

    plsc.subcore_barrier()

    def _fire_group(t, n):
        gb = base_w + n * _GE
        pltpu.sync_copy(nrm_hbm.at[pl.ds(gb, _GE)], nrm_m[t])
        for b in range(_GB):
            pltpu.sync_copy(src_hbm.at[pl.ds(gb + b * _EB, _EB)], idx[t][b])
            pltpu.sync_copy(dst_hbm.at[pl.ds(gb + b * _EB, _EB)], dstv[t][b])
        for b in range(_GB):
            for k in range(_EB // 16):
                sl = pl.ds(k * 16, 16)
                idx[t][b][sl] = idx[t][b][sl] + roff
                d = dstv[t][b][sl] - doff
                ok = (d >= 0) & (d < _HALF)
                dstv[t][b][sl] = jnp.where(ok, d, _HALF)
        for b in range(_GB):
            pltpu.async_copy(
                table_hbm.at[idx[t][b]], rows.at[pl.ds((t * _GB + b) * _EB, _EB), :], gsem[t])

    def _process_group(t):
        for b in range(_GB):
            pltpu.make_async_copy(
                table_hbm.at[idx[t][b]],
                rows.at[pl.ds((t * _GB + b) * _EB, _EB), :], gsem[t]).wait()
        for b in range(_GB):
            j3 = t * _GB + b

            def _scale(g2, inner):
                nv16 = nrm_m[t][pl.ds(b * _EB + g2 * 16, 16)]
                for i in range(16):
                    nv = lax.gather(
                        nv16, jnp.full((16, 1), i, jnp.int32), _DNUMS,
                        slice_sizes=(1,),
                        mode=lax.GatherScatterMode.PROMISE_IN_BOUNDS)
                    j = g2 * 16 + i
                    for k in range(8):
                        sl = pl.ds(k * 16, 16)
                        rows[j3 * _EB + j, sl] = rows[j3 * _EB + j, sl] * nv
                return inner

            lax.fori_loop(0, _EB // 16, _scale, 0)
            pltpu.async_copy(rows.at[pl.ds(j3 * _EB, _EB), :],
                             agg.at[dstv[t][b]], ssem[t],
                             add=True)

    def _drain_scatters(t):
        for b in range(_GB):
            pltpu.make_async_copy(
                rows.at[pl.ds((t * _GB + b) * _EB, _EB), :],
                agg.at[dstv[t][b]], ssem[t]).wait()

    _fire_group(0, 0)

    def _pair(gp, carry):
        for s in range(2):
            g = gp * 2 + s
            n = g + 1

            @pl.when(n < _NGRP)
            def _fire():
                @pl.when(g >= 1)
                def _wait_prev():
                    _drain_scatters(1 - s)

                _fire_group(1 - s, n)

            _process_group(s)
        return carry

    lax.fori_loop(0, _NGRP // 2, _pair, 0)
    _process_group(0)
    _drain_scatters(0)
    _drain_scatters(1)
    plsc.subcore_barrier()

    sl = pl.ds(sid * _ROWS_PW, _ROWS_PW)
    pltpu.sync_copy(agg.at[sl], out_hbm.at[cid, sl])


def _aggregate(table, src, dst2, nrm):
    mesh = plsc.VectorSubcoreMesh(core_axis_name="c", subcore_axis_name="s")
    fn = pl.kernel(
        _sc_body,
        out_type=jax.ShapeDtypeStruct((2, _HALF, _D), jnp.float32),
        mesh=mesh,
        scratch_types=(
            [
                pltpu.VMEM_SHARED((_AGG, _D), jnp.float32),
                pltpu.VMEM((40, _D), jnp.float32),
                pltpu.VMEM((2 * _GB * _EB, _D), jnp.float32),
            ]
            + [pltpu.VMEM((_EB,), jnp.int32) for _ in range(2 * _GB)]
            + [pltpu.VMEM((_EB,), jnp.int32) for _ in range(2 * _GB)]
            + [pltpu.VMEM((_GE,), jnp.float32) for _ in range(2)]
            + [pltpu.SemaphoreType.DMA for _ in range(4)]
        ),
    )
    return fn(table, src, dst2, nrm)


def _epilogue_body(p_ref, s_ref, g1_ref, b1_ref, g2_ref, b2_ref,
                   fin_ref, fout_ref, o_ref):
    t = p_ref[...] + s_ref[...]
    h = _gelu(_ln(t, g1_ref[...], b1_ref[...]))
    y = jnp.dot(_ln(h, g2_ref[...], b2_ref[...]), fin_ref[...],
                preferred_element_type=jnp.float32)
    y = jnp.dot(_gelu(y), fout_ref[...], preferred_element_type=jnp.float32)
    o_ref[...] = h + y


def _epilogue(partials, selfpart, g1, b1, g2, b2, fin_t, fout_t):
    vec = pl.BlockSpec((1, _D), lambda i: (0, 0))
    return pl.pallas_call(
        _epilogue_body,
        grid=(_N // _BN,),
        in_specs=[
            pl.BlockSpec((_BN, _D), lambda i: (i, 0)),
            pl.BlockSpec((_BN, _D), lambda i: (i, 0)),
            vec, vec, vec, vec,
            pl.BlockSpec((_D, _INNER), lambda i: (0, 0)),
            pl.BlockSpec((_INNER, _D), lambda i: (0, 0)),
        ],
        out_specs=pl.BlockSpec((_BN, _D), lambda i: (i, 0)),
        out_shape=jax.ShapeDtypeStruct((_N, _D), jnp.float32),
    )(partials, selfpart, g1, b1, g2, b2, fin_t, fout_t)


def kernel(x, source, destination, normalization, relation_weights,
           self_loop_W, ln1_g, ln1_b, ln2_g, ln2_b, ffn_in_W, ffn_out_W):
    w9 = jnp.concatenate([relation_weights, self_loop_W.T[None]], axis=0)
    table, selfpart = _transform(x, w9)
    partials = _aggregate(
        table.reshape(_R * _N, _D),
        source.reshape(-1),
        destination.reshape(-1),
        normalization.reshape(-1),
    )
    return _epilogue(
        partials.reshape(2 * _HALF, _D), selfpart,
        ln1_g.reshape(1, _D), ln1_b.reshape(1, _D),
        ln2_g.reshape(1, _D), ln2_b.reshape(1, _D),
        ffn_in_W.T, ffn_out_W.T,
    )


# speedup vs baseline: 3.4891x; 1.2306x over previous
import jax
import jax.numpy as jnp
from jax import lax
from jax.experimental import pallas as pl
from jax.experimental.pallas import tpu as pltpu
from jax.experimental.pallas import tpu_sc as plsc

_N = 10000
_D = 128
_R = 8
_E = 40000
_INNER = 512
_EPS = 1e-5

_NS = 16
_EPW = (_R * _E) // _NS
_EB = 80
_GB = 2
_GE = _EB * _GB
_NGRP = _EPW // _GE
_HALF = 5120
_AGG = _HALF + 8
_ROWS_PW = _HALF // _NS
_BN = 400


def _gelu(h):
    return 0.5 * h * (1.0 + lax.erf(h * 0.7071067811865476))


def _ln(h, g, b):
    m = jnp.mean(h, axis=-1, keepdims=True)
    v = jnp.mean((h - m) ** 2, axis=-1, keepdims=True)
    return (h - m) * lax.rsqrt(v + _EPS) * g + b


def _transform_body(x_ref, w_ref, tab_ref, self_ref):
    r = pl.program_id(1)
    xb = x_ref[...]
    acc = jnp.dot(xb, w_ref[0], preferred_element_type=jnp.float32)

    @pl.when(r < _R)
    def _():
        tab_ref[0] = acc

    @pl.when(r == _R)
    def _():
        self_ref[...] = acc + xb


def _transform(x, w9):
    return pl.pallas_call(
        _transform_body,
        grid=(_N // _BN, _R + 1),
        in_specs=[
            pl.BlockSpec((_BN, _D), lambda i, r: (i, 0)),
            pl.BlockSpec((1, _D, _D), lambda i, r: (r, 0, 0)),
        ],
        out_specs=[
            pl.BlockSpec((1, _BN, _D), lambda i, r: (jnp.minimum(r, _R - 1), i, 0)),
            pl.BlockSpec((_BN, _D), lambda i, r: (i, 0)),
        ],
        out_shape=[
            jax.ShapeDtypeStruct((_R, _N, _D), jnp.float32),
            jax.ShapeDtypeStruct((_N, _D), jnp.float32),
        ],
    )(x, w9)


_DNUMS = lax.GatherDimensionNumbers(
    offset_dims=(), collapsed_slice_dims=(0,), start_index_map=(0,))


def _sc_body(table_hbm, meta_hbm, out_hbm, *scr):
    agg, zbuf, rows = scr[0], scr[1], scr[2]
    ng = 2 * _GB
    idx = (scr[3:3 + _GB], scr[3 + _GB:3 + ng])
    dstv = (scr[3 + ng:3 + ng + _GB], scr[3 + ng + _GB:3 + 2 * ng])
    mbuf = (scr[3 + 2 * ng], scr[4 + 2 * ng])
    gsem = (scr[5 + 2 * ng], scr[6 + 2 * ng])
    ssem = (scr[7 + 2 * ng], scr[8 + 2 * ng])
    cid = lax.axis_index("c")
    sid = lax.axis_index("s")
    roff = (sid // 2) * _N
    doff = cid * _HALF
    base_w = sid * _EPW

    zero = jnp.zeros((16,), jnp.float32)

    def _zfill(j, carry):
        for k in range(8):
            zbuf[j, pl.ds(k * 16, 16)] = zero
        return carry

    lax.fori_loop(0, 40, _zfill, 0)
    for z in range(_ROWS_PW // 40):
        pltpu.sync_copy(zbuf, agg.at[pl.ds(sid * _ROWS_PW + z * 40, 40)])

    @pl.when(sid == 0)
    def _zdump():
        pltpu.sync_copy(zbuf.at[pl.ds(0, 8)], agg.at[pl.ds(_HALF, 8)])

    plsc.subcore_barrier()

    def _fire_group(t, n):
        pltpu.sync_copy(meta_hbm.at[sid * _NGRP + n], mbuf[t])
        for b in range(_GB):
            for k in range(_EB // 16):
                sl = pl.ds(k * 16, 16)
                slm = pl.ds(b * _EB + k * 16, 16)
                idx[t][b][sl] = mbuf[t][0, slm] + roff
                d = mbuf[t][1, slm] - doff
                ok = (d >= 0) & (d < _HALF)
                dstv[t][b][sl] = jnp.where(ok, d, _HALF)
        for b in range(_GB):
            pltpu.async_copy(
                table_hbm.at[idx[t][b]], rows.at[pl.ds((t * _GB + b) * _EB, _EB), :], gsem[t])

    def _process_group(t):
        for b in range(_GB):
            pltpu.make_async_copy(
                table_hbm.at[idx[t][b]],
                rows.at[pl.ds((t * _GB + b) * _EB, _EB), :], gsem[t]).wait()
        for b in range(_GB):
            j3 = t * _GB + b

            def _scale(g2, inner):
                nv16 = lax.bitcast_convert_type(
                    mbuf[t][2, pl.ds(b * _EB + g2 * 16, 16)], jnp.float32)
                for i in range(16):
                    nv = lax.gather(
                        nv16, jnp.full((16, 1), i, jnp.int32), _DNUMS,
                        slice_sizes=(1,),
                        mode=lax.GatherScatterMode.PROMISE_IN_BOUNDS)
                    j = g2 * 16 + i
                    for k in range(8):
                        sl = pl.ds(k * 16, 16)
                        rows[j3 * _EB + j, sl] = rows[j3 * _EB + j, sl] * nv
                return inner

            lax.fori_loop(0, _EB // 16, _scale, 0)
            pltpu.async_copy(rows.at[pl.ds(j3 * _EB, _EB), :],
                             agg.at[dstv[t][b]], ssem[t],
                             add=True)

    def _drain_scatters(t):
        for b in range(_GB):
            pltpu.make_async_copy(
                rows.at[pl.ds((t * _GB + b) * _EB, _EB), :],
                agg.at[dstv[t][b]], ssem[t]).wait()

    _fire_group(0, 0)

    def _pair(gp, carry):
        for s in range(2):
            g = gp * 2 + s
            n = g + 1

            @pl.when(n < _NGRP)
            def _fire():
                @pl.when(g >= 1)
                def _wait_prev():
                    _drain_scatters(1 - s)

                _fire_group(1 - s, n)

            _process_group(s)
        return carry

    lax.fori_loop(0, _NGRP // 2, _pair, 0)
    _process_group(0)
    _drain_scatters(0)
    _drain_scatters(1)
    plsc.subcore_barrier()

    sl = pl.ds(sid * _ROWS_PW, _ROWS_PW)
    pltpu.sync_copy(agg.at[sl], out_hbm.at[cid, sl])


def _aggregate(table, meta):
    mesh = plsc.VectorSubcoreMesh(core_axis_name="c", subcore_axis_name="s")
    fn = pl.kernel(
        _sc_body,
        out_type=jax.ShapeDtypeStruct((2, _HALF, _D), jnp.float32),
        mesh=mesh,
        scratch_types=(
            [
                pltpu.VMEM_SHARED((_AGG, _D), jnp.float32),
                pltpu.VMEM((40, _D), jnp.float32),
                pltpu.VMEM((2 * _GB * _EB, _D), jnp.float32),
            ]
            + [pltpu.VMEM((_EB,), jnp.int32) for _ in range(2 * _GB)]
            + [pltpu.VMEM((_EB,), jnp.int32) for _ in range(2 * _GB)]
            + [pltpu.VMEM((3, _GE), jnp.int32) for _ in range(2)]
            + [pltpu.SemaphoreType.DMA for _ in range(4)]
        ),
    )
    return fn(table, meta)


def _epilogue_body(p_ref, s_ref, g1_ref, b1_ref, g2_ref, b2_ref,
                   fin_ref, fout_ref, o_ref):
    t = p_ref[...] + s_ref[...]
    h = _gelu(_ln(t, g1_ref[...], b1_ref[...]))
    y = jnp.dot(_ln(h, g2_ref[...], b2_ref[...]), fin_ref[...],
                preferred_element_type=jnp.float32)
    y = jnp.dot(_gelu(y), fout_ref[...], preferred_element_type=jnp.float32)
    o_ref[...] = h + y


def _epilogue(partials, selfpart, g1, b1, g2, b2, fin_t, fout_t):
    vec = pl.BlockSpec((1, _D), lambda i: (0, 0))
    return pl.pallas_call(
        _epilogue_body,
        grid=(_N // _BN,),
        in_specs=[
            pl.BlockSpec((_BN, _D), lambda i: (i, 0)),
            pl.BlockSpec((_BN, _D), lambda i: (i, 0)),
            vec, vec, vec, vec,
            pl.BlockSpec((_D, _INNER), lambda i: (0, 0)),
            pl.BlockSpec((_INNER, _D), lambda i: (0, 0)),
        ],
        out_specs=pl.BlockSpec((_BN, _D), lambda i: (i, 0)),
        out_shape=jax.ShapeDtypeStruct((_N, _D), jnp.float32),
    )(partials, selfpart, g1, b1, g2, b2, fin_t, fout_t)


def kernel(x, source, destination, normalization, relation_weights,
           self_loop_W, ln1_g, ln1_b, ln2_g, ln2_b, ffn_in_W, ffn_out_W):
    w9 = jnp.concatenate([relation_weights, self_loop_W.T[None]], axis=0)
    table, selfpart = _transform(x, w9)
    meta = jnp.stack(
        [
            source.reshape(_NS, _NGRP, _GE),
            destination.reshape(_NS, _NGRP, _GE),
            lax.bitcast_convert_type(
                normalization.reshape(_NS, _NGRP, _GE), jnp.int32),
        ],
        axis=2,
    ).reshape(_NS * _NGRP, 3, _GE)
    partials = _aggregate(table.reshape(_R * _N, _D), meta)
    return _epilogue(
        partials.reshape(2 * _HALF, _D), selfpart,
        ln1_g.reshape(1, _D), ln1_b.reshape(1, _D),
        ln2_g.reshape(1, _D), ln2_b.reshape(1, _D),
        ffn_in_W.T, ffn_out_W.T,
    )


# speedup vs baseline: 3.5628x; 1.0211x over previous
import jax
import jax.numpy as jnp
from jax import lax
from jax.experimental import pallas as pl
from jax.experimental.pallas import tpu as pltpu
from jax.experimental.pallas import tpu_sc as plsc

_N = 10000
_D = 128
_R = 8
_E = 40000
_INNER = 512
_EPS = 1e-5

_NS = 16
_EPW = (_R * _E) // _NS
_EB = 80
_GB = 2
_GE = _EB * _GB
_NGRP = _EPW // _GE
_HALF = 5120
_AGG = _HALF + 8
_ROWS_PW = _HALF // _NS
_BN = 400


def _gelu(h):
    return 0.5 * h * (1.0 + lax.erf(h * 0.7071067811865476))


def _ln(h, g, b):
    m = jnp.mean(h, axis=-1, keepdims=True)
    v = jnp.mean((h - m) ** 2, axis=-1, keepdims=True)
    return (h - m) * lax.rsqrt(v + _EPS) * g + b


def _transform_body(x_ref, w_ref, tab_ref, self_ref):
    r = pl.program_id(1)
    xb = x_ref[...]
    acc = jnp.dot(xb, w_ref[0], preferred_element_type=jnp.float32)

    @pl.when(r < _R)
    def _():
        tab_ref[0] = acc

    @pl.when(r == _R)
    def _():
        self_ref[...] = acc + xb


def _transform(x, w9):
    return pl.pallas_call(
        _transform_body,
        grid=(_N // _BN, _R + 1),
        in_specs=[
            pl.BlockSpec((_BN, _D), lambda i, r: (i, 0)),
            pl.BlockSpec((1, _D, _D), lambda i, r: (r, 0, 0)),
        ],
        out_specs=[
            pl.BlockSpec((1, _BN, _D), lambda i, r: (jnp.minimum(r, _R - 1), i, 0)),
            pl.BlockSpec((_BN, _D), lambda i, r: (i, 0)),
        ],
        out_shape=[
            jax.ShapeDtypeStruct((_R, _N, _D), jnp.float32),
            jax.ShapeDtypeStruct((_N, _D), jnp.float32),
        ],
    )(x, w9)


_DNUMS = lax.GatherDimensionNumbers(
    offset_dims=(), collapsed_slice_dims=(0,), start_index_map=(0,))


def _sc_body(table_hbm, meta_hbm, out_hbm, *scr):
    agg, zbuf, rows = scr[0], scr[1], scr[2]
    ng = 2 * _GB
    idx = (scr[3:3 + _GB], scr[3 + _GB:3 + ng])
    dstv = (scr[3 + ng:3 + ng + _GB], scr[3 + ng + _GB:3 + 2 * ng])
    mbuf = (scr[3 + 2 * ng], scr[4 + 2 * ng])
    gsem = (scr[5 + 2 * ng], scr[6 + 2 * ng])
    ssem = (scr[7 + 2 * ng], scr[8 + 2 * ng])
    msem = scr[9 + 2 * ng]
    cid = lax.axis_index("c")
    sid = lax.axis_index("s")
    roff = (sid // 2) * _N
    doff = cid * _HALF
    base_w = sid * _EPW

    zero = jnp.zeros((16,), jnp.float32)

    def _zfill(j, carry):
        for k in range(8):
            zbuf[j, pl.ds(k * 16, 16)] = zero
        return carry

    lax.fori_loop(0, 40, _zfill, 0)
    for z in range(_ROWS_PW // 40):
        pltpu.sync_copy(zbuf, agg.at[pl.ds(sid * _ROWS_PW + z * 40, 40)])

    @pl.when(sid == 0)
    def _zdump():
        pltpu.sync_copy(zbuf.at[pl.ds(0, 8)], agg.at[pl.ds(_HALF, 8)])

    plsc.subcore_barrier()

    def _meta_block_src(blk):
        return meta_hbm.at[pl.ds((sid * (_NGRP + 1) + 2 * blk), 2), :, :]

    def _fire_group(t, n, v, g01):
        for b in range(_GB):
            for k in range(_EB // 16):
                sl = pl.ds(k * 16, 16)
                slm = pl.ds(b * _EB + k * 16, 16)
                idx[t][b][sl] = mbuf[v][g01, 0, slm] + roff
                d = mbuf[v][g01, 1, slm] - doff
                ok = (d >= 0) & (d < _HALF)
                dstv[t][b][sl] = jnp.where(ok, d, _HALF)
        for b in range(_GB):
            pltpu.async_copy(
                table_hbm.at[idx[t][b]],
                rows.at[pl.ds((t * _GB + b) * _EB, _EB), :], gsem[t])

    def _process_group(t, v, g01):
        for b in range(_GB):
            pltpu.make_async_copy(
                table_hbm.at[idx[t][b]],
                rows.at[pl.ds((t * _GB + b) * _EB, _EB), :], gsem[t]).wait()
        for b in range(_GB):
            j3 = t * _GB + b

            def _scale(g2, inner):
                nv16 = lax.bitcast_convert_type(
                    mbuf[v][g01, 2, pl.ds(b * _EB + g2 * 16, 16)], jnp.float32)
                for i in range(16):
                    nv = lax.gather(
                        nv16, jnp.full((16, 1), i, jnp.int32), _DNUMS,
                        slice_sizes=(1,),
                        mode=lax.GatherScatterMode.PROMISE_IN_BOUNDS)
                    j = g2 * 16 + i
                    for k in range(8):
                        sl = pl.ds(k * 16, 16)
                        rows[j3 * _EB + j, sl] = rows[j3 * _EB + j, sl] * nv
                return inner

            lax.fori_loop(0, _EB // 16, _scale, 0)
            pltpu.async_copy(rows.at[pl.ds(j3 * _EB, _EB), :],
                             agg.at[dstv[t][b]], ssem[t],
                             add=True)

    def _drain_scatters(t):
        for b in range(_GB):
            pltpu.make_async_copy(
                rows.at[pl.ds((t * _GB + b) * _EB, _EB), :],
                agg.at[dstv[t][b]], ssem[t]).wait()

    pltpu.sync_copy(_meta_block_src(0), mbuf[0])
    _fire_group(0, 0, 0, 0)

    def _quad(q, carry):
        for w in range(4):
            n = q * 4 + w
            s = w % 2
            v = (w // 2) % 2

            @pl.when(n >= 1)
            def _wait_prev():
                _drain_scatters(1 - s)

            if w % 2 == 1:
                pltpu.make_async_copy(
                    _meta_block_src((n + 1) // 2), mbuf[1 - v], msem).wait()
                _fire_group(1 - s, n + 1, 1 - v, 0)
            else:
                _fire_group(1 - s, n + 1, v, 1)
                pltpu.async_copy(
                    _meta_block_src(n // 2 + 1), mbuf[1 - v], msem)

            _process_group(s, v, w % 2)
        return carry

    lax.fori_loop(0, (_NGRP - 1) // 4, _quad, 0)
    _process_group(0, 0, 0)
    _drain_scatters(0)
    _drain_scatters(1)
    plsc.subcore_barrier()

    sl = pl.ds(sid * _ROWS_PW, _ROWS_PW)
    pltpu.sync_copy(agg.at[sl], out_hbm.at[cid, sl])


def _aggregate(table, meta):
    mesh = plsc.VectorSubcoreMesh(core_axis_name="c", subcore_axis_name="s")
    fn = pl.kernel(
        _sc_body,
        out_type=jax.ShapeDtypeStruct((2, _HALF, _D), jnp.float32),
        mesh=mesh,
        scratch_types=(
            [
                pltpu.VMEM_SHARED((_AGG, _D), jnp.float32),
                pltpu.VMEM((40, _D), jnp.float32),
                pltpu.VMEM((2 * _GB * _EB, _D), jnp.float32),
            ]
            + [pltpu.VMEM((_EB,), jnp.int32) for _ in range(2 * _GB)]
            + [pltpu.VMEM((_EB,), jnp.int32) for _ in range(2 * _GB)]
            + [pltpu.VMEM((2, 3, _GE), jnp.int32) for _ in range(2)]
            + [pltpu.SemaphoreType.DMA for _ in range(5)]
        ),
    )
    return fn(table, meta)


def _epilogue_body(p_ref, s_ref, g1_ref, b1_ref, g2_ref, b2_ref,
                   fin_ref, fout_ref, o_ref):
    t = p_ref[...] + s_ref[...]
    h = _gelu(_ln(t, g1_ref[...], b1_ref[...]))
    y = jnp.dot(_ln(h, g2_ref[...], b2_ref[...]), fin_ref[...],
                preferred_element_type=jnp.float32)
    y = jnp.dot(_gelu(y), fout_ref[...], preferred_element_type=jnp.float32)
    o_ref[...] = h + y


def _epilogue(partials, selfpart, g1, b1, g2, b2, fin_t, fout_t):
    vec = pl.BlockSpec((1, _D), lambda i: (0, 0))
    return pl.pallas_call(
        _epilogue_body,
        grid=(_N // _BN,),
        in_specs=[
            pl.BlockSpec((_BN, _D), lambda i: (i, 0)),
            pl.BlockSpec((_BN, _D), lambda i: (i, 0)),
            vec, vec, vec, vec,
            pl.BlockSpec((_D, _INNER), lambda i: (0, 0)),
            pl.BlockSpec((_INNER, _D), lambda i: (0, 0)),
        ],
        out_specs=pl.BlockSpec((_BN, _D), lambda i: (i, 0)),
        out_shape=jax.ShapeDtypeStruct((_N, _D), jnp.float32),
    )(partials, selfpart, g1, b1, g2, b2, fin_t, fout_t)


def kernel(x, source, destination, normalization, relation_weights,
           self_loop_W, ln1_g, ln1_b, ln2_g, ln2_b, ffn_in_W, ffn_out_W):
    w9 = jnp.concatenate([relation_weights, self_loop_W.T[None]], axis=0)
    table, selfpart = _transform(x, w9)
    meta = jnp.stack(
        [
            source.reshape(_NS, _NGRP, _GE),
            destination.reshape(_NS, _NGRP, _GE),
            lax.bitcast_convert_type(
                normalization.reshape(_NS, _NGRP, _GE), jnp.int32),
        ],
        axis=2,
    )
    meta = jnp.pad(meta, ((0, 0), (0, 1), (0, 0), (0, 0)))
    meta = meta.reshape(_NS * (_NGRP + 1), 3, _GE)
    partials = _aggregate(table.reshape(_R * _N, _D), meta)
    return _epilogue(
        partials.reshape(2 * _HALF, _D), selfpart,
        ln1_g.reshape(1, _D), ln1_b.reshape(1, _D),
        ln2_g.reshape(1, _D), ln2_b.reshape(1, _D),
        ffn_in_W.T, ffn_out_W.T,
    )
